# Initial kernel scaffold; baseline (speedup 1.0000x reference)
import jax
import jax.numpy as jnp
from jax.experimental import pallas as pl


def kernel(embeddings, instance_ids):
    raise NotImplementedError("write your pallas kernel here")


# speedup vs baseline: 7.0507x; 7.0507x over previous
import functools

import jax
import jax.numpy as jnp
from jax.experimental import pallas as pl
from jax.experimental.pallas import tpu as pltpu

_DELTA_V = 0.5
_DELTA_D = 1.5
_ALPHA = 1.0
_BETA = 1.0
_GAMMA = 0.001
_K = 64


_CHUNK = 2048


def _dot(a, b, dims):
    return jax.lax.dot_general(
        a, b, (dims, ((), ())),
        preferred_element_type=jnp.float32,
        precision=jax.lax.Precision.HIGHEST)


def _row_body(ids_ref, e_ref, out_ref, oh_ref):
    B = e_ref.shape[1]
    D = e_ref.shape[2]
    nch = B // _CHUNK

    iota_k = jax.lax.broadcasted_iota(jnp.int32, (_K, _CHUNK), 0)
    sums = jnp.zeros((_K, D), jnp.float32)
    counts = jnp.zeros((_K,), jnp.float32)
    for c in range(nch):
        idc = ids_ref[0, :, c * _CHUNK:(c + 1) * _CHUNK]
        ohc = jnp.where(idc == iota_k, 1.0, 0.0)
        oh_ref[:, c * _CHUNK:(c + 1) * _CHUNK] = ohc
        Ec = e_ref[0, c * _CHUNK:(c + 1) * _CHUNK, :]
        sums = sums + _dot(ohc, Ec, ((1,), (0,)))
        counts = counts + jnp.sum(ohc, axis=1)
    cnt = jnp.maximum(counts, 1.0)
    inv = 1.0 / cnt
    means = sums * inv[:, None]

    var_acc = jnp.zeros((), jnp.float32)
    for c in range(nch):
        ohc = oh_ref[:, c * _CHUNK:(c + 1) * _CHUNK]
        Ec = e_ref[0, c * _CHUNK:(c + 1) * _CHUNK, :]
        mg = _dot(ohc, means, ((0,), (0,)))
        invg = _dot(ohc, inv[:, None], ((0,), (0,)))[:, 0]
        diff = Ec - mg
        ssq = jnp.sum(diff * diff, axis=1) + 1e-12
        dist = jnp.sqrt(ssq)
        h = jnp.maximum(dist - _DELTA_V, 0.0)
        var_acc = var_acc + jnp.sum(h * h * invg)
    var_loss = var_acc / _K

    md = means[:, None, :] - means[None, :, :]
    sq = jnp.sum(md * md, axis=-1)
    ii = jax.lax.broadcasted_iota(jnp.int32, (_K, _K), 0)
    jj = jax.lax.broadcasted_iota(jnp.int32, (_K, _K), 1)
    offdiag = ii != jj
    pd = jnp.sqrt(jnp.where(offdiag, sq, 1.0))
    hp = jnp.maximum(2.0 * _DELTA_D - pd, 0.0)
    num_pairs = _K * (_K - 1) / 2.0
    dist_loss = jnp.sum(jnp.where(offdiag, hp * hp, 0.0)) / (2.0 * num_pairs)

    reg_loss = jnp.mean(jnp.sqrt(jnp.sum(means * means, axis=1) + 1e-12))

    lane = jax.lax.broadcasted_iota(jnp.int32, (1, 128), 1)
    vec = jnp.where(lane == 0, var_loss,
                    jnp.where(lane == 1, dist_loss,
                              jnp.where(lane == 2, reg_loss, 0.0)))
    out_ref[0] = vec


@functools.partial(jax.jit, static_argnames=())
def kernel(embeddings, instance_ids):
    Bt, N, D = embeddings.shape
    ids3 = instance_ids.reshape(Bt, 1, N).astype(jnp.int32)
    out = pl.pallas_call(
        _row_body,
        grid=(Bt,),
        in_specs=[
            pl.BlockSpec((1, 1, N), lambda r: (r, 0, 0)),
            pl.BlockSpec((1, N, D), lambda r: (r, 0, 0)),
        ],
        out_specs=pl.BlockSpec((1, 1, 128), lambda r: (r, 0, 0)),
        out_shape=jax.ShapeDtypeStruct((Bt, 1, 128), jnp.float32),
        scratch_shapes=[pltpu.VMEM((_K, N), jnp.float32)],
    )(ids3, embeddings)
    var_loss = jnp.mean(out[:, 0, 0])
    dist_loss = jnp.mean(out[:, 0, 1])
    reg_loss = jnp.mean(out[:, 0, 2])
    total = _ALPHA * var_loss + _BETA * dist_loss + _GAMMA * reg_loss
    return (total, var_loss, dist_loss, reg_loss)


# speedup vs baseline: 100.3890x; 14.2382x over previous
import functools

import jax
import jax.numpy as jnp
from jax.experimental import pallas as pl
from jax.experimental.pallas import tpu as pltpu

_DELTA_V = 0.5
_DELTA_D = 1.5
_ALPHA = 1.0
_BETA = 1.0
_GAMMA = 0.001
_K = 64

_CHUNK = 8192


def _dot(a, b, dims):
    return jax.lax.dot_general(
        a, b, (dims, ((), ())), preferred_element_type=jnp.float32)


def _row_body(ids_ref, et_ref, out_ref, oh_ref):
    B = et_ref.shape[2]
    D = et_ref.shape[1]
    nch = B // _CHUNK

    iota_k = jax.lax.broadcasted_iota(jnp.int32, (_K, _CHUNK), 0)
    ones_row = jnp.ones((1, _CHUNK), jnp.float32)
    sums_t = jnp.zeros((D, _K), jnp.float32)
    counts = jnp.zeros((1, _K), jnp.float32)
    for c in range(nch):
        sl = slice(c * _CHUNK, (c + 1) * _CHUNK)
        idc = ids_ref[0, :, sl]
        ohc = jnp.where(idc == iota_k, 1.0, 0.0)
        oh_ref[:, sl] = ohc
        etc = et_ref[0, :, sl]
        sums_t = sums_t + _dot(etc, ohc, ((1,), (1,)))
        counts = counts + _dot(ones_row, ohc, ((1,), (1,)))
    cnt = jnp.maximum(counts, 1.0)
    inv = 1.0 / cnt
    means_t = sums_t * inv

    var_acc = jnp.zeros((), jnp.float32)
    for c in range(nch):
        sl = slice(c * _CHUNK, (c + 1) * _CHUNK)
        ohc = oh_ref[:, sl]
        mg_t = _dot(means_t, ohc, ((1,), (0,)))
        invg = _dot(inv, ohc, ((1,), (0,)))
        etc = et_ref[0, :, sl]
        diff = etc - mg_t
        ssq = jnp.sum(diff * diff, axis=0) + 1e-12
        dist = jnp.sqrt(ssq)
        h = jnp.maximum(dist - _DELTA_V, 0.0)
        var_acc = var_acc + jnp.sum(h * h * invg[0])
    var_loss = var_acc / _K

    md = means_t[:, :, None] - means_t[:, None, :]
    sq = jnp.sum(md * md, axis=0)
    ii = jax.lax.broadcasted_iota(jnp.int32, (_K, _K), 0)
    jj = jax.lax.broadcasted_iota(jnp.int32, (_K, _K), 1)
    offdiag = ii != jj
    pd = jnp.sqrt(jnp.where(offdiag, sq, 1.0))
    hp = jnp.maximum(2.0 * _DELTA_D - pd, 0.0)
    num_pairs = _K * (_K - 1) / 2.0
    dist_loss = jnp.sum(jnp.where(offdiag, hp * hp, 0.0)) / (2.0 * num_pairs)

    reg_loss = jnp.mean(jnp.sqrt(jnp.sum(means_t * means_t, axis=0) + 1e-12))

    lane = jax.lax.broadcasted_iota(jnp.int32, (1, 128), 1)
    vec = jnp.where(lane == 0, var_loss,
                    jnp.where(lane == 1, dist_loss,
                              jnp.where(lane == 2, reg_loss, 0.0)))
    out_ref[0] = vec


@functools.partial(jax.jit, static_argnames=())
def kernel(embeddings, instance_ids):
    Bt, N, D = embeddings.shape
    ids3 = instance_ids.reshape(Bt, 1, N).astype(jnp.int32)
    emb_t = embeddings.transpose(0, 2, 1)
    out = pl.pallas_call(
        _row_body,
        grid=(Bt,),
        in_specs=[
            pl.BlockSpec((1, 1, N), lambda r: (r, 0, 0)),
            pl.BlockSpec((1, D, N), lambda r: (r, 0, 0)),
        ],
        out_specs=pl.BlockSpec((1, 1, 128), lambda r: (r, 0, 0)),
        out_shape=jax.ShapeDtypeStruct((Bt, 1, 128), jnp.float32),
        scratch_shapes=[pltpu.VMEM((_K, N), jnp.float32)],
    )(ids3, emb_t)
    var_loss = jnp.mean(out[:, 0, 0])
    dist_loss = jnp.mean(out[:, 0, 1])
    reg_loss = jnp.mean(out[:, 0, 2])
    total = _ALPHA * var_loss + _BETA * dist_loss + _GAMMA * reg_loss
    return (total, var_loss, dist_loss, reg_loss)
